# baseline (speedup 1.0000x reference)
import functools

import jax
import jax.numpy as jnp
from jax import lax
from jax.experimental import pallas as pl
from jax.experimental.pallas import tpu as pltpu

_EPS = 1e-5
_VMEM_LIMIT = 56 * 1024 * 1024


def _build_taps(src, col_ref, b, K, H, W, Wq, C, cast):
    pad = (K - 1) // 2
    col_ref[b, :, 0:pad, :] = jnp.zeros((H, pad, K * C), jnp.bfloat16)
    col_ref[b, :, pad + W:Wq, :] = jnp.zeros((H, Wq - pad - W, K * C),
                                             jnp.bfloat16)
    for di in range(K):
        r0 = max(0, pad - di)
        r1 = min(H, H + pad - di)
        if r0 > 0:
            col_ref[b, 0:r0, pad:pad + W, di * C:(di + 1) * C] = jnp.zeros(
                (r0, W, C), jnp.bfloat16)
        if r1 < H:
            col_ref[b, r1:H, pad:pad + W, di * C:(di + 1) * C] = jnp.zeros(
                (H - r1, W, C), jnp.bfloat16)
        patch = src[r0 + di - pad:r1 + di - pad]
        if cast:
            patch = patch.astype(jnp.bfloat16)
        col_ref[b, r0:r1, pad:pad + W, di * C:(di + 1) * C] = patch


def _conv_epilogue(col_ref, w_ref, p_ref, B, K, H, W, Wq, C):
    p_ref[...] = jnp.dot(
        col_ref[...].reshape(B * H * Wq, K * C), w_ref[...],
        preferred_element_type=jnp.float32).reshape(B, H, Wq, K * C)
    y = p_ref[:, :, 0:W, 0:C]
    for dj in range(1, K):
        y = y + p_ref[:, :, dj:dj + W, dj * C:(dj + 1) * C]
    return y


def _tile_stats(y, stat_ref, m_tile, C):
    acc = y.reshape(m_tile, C)
    stat_ref[0, 0:1, :] = jnp.sum(acc, axis=0, keepdims=True)
    stat_ref[0, 1:2, :] = jnp.sum(acc * acc, axis=0, keepdims=True)


def _conv1_kernel(x_ref, w_ref, y_ref, stat_ref, xb_ref, col_ref, p_ref,
                  *, B, K, H, W, Wq, C):
    xb_ref[...] = x_ref[...].astype(jnp.bfloat16)
    for b in range(B):
        _build_taps(xb_ref[b], col_ref, b, K, H, W, Wq, C, cast=False)
    y = _conv_epilogue(col_ref, w_ref, p_ref, B, K, H, W, Wq, C)
    y_ref[...] = y.astype(jnp.bfloat16)
    _tile_stats(y, stat_ref, B * H * W, C)


def _mid_kernel(y1_ref, x_ref, w_ref, s1_ref, t1_ref,
                y2_ref, x1_ref, stat_ref, col_ref, p_ref,
                *, B, K, H, W, Wq, C):
    x1 = (jnp.maximum(y1_ref[...].astype(jnp.float32) * s1_ref[...]
                      + t1_ref[...], 0.0) + x_ref[...])
    x1_ref[...] = x1.astype(jnp.bfloat16)
    for b in range(B):
        _build_taps(x1_ref[b], col_ref, b, K, H, W, Wq, C, cast=False)
    y = _conv_epilogue(col_ref, w_ref, p_ref, B, K, H, W, Wq, C)
    y2_ref[...] = y.astype(jnp.bfloat16)
    _tile_stats(y, stat_ref, B * H * W, C)


def _final_kernel(y2_ref, x1_ref, s2_ref, t2_ref, out_ref):
    out_ref[...] = (jnp.maximum(y2_ref[...].astype(jnp.float32) * s2_ref[...]
                                + t2_ref[...], 0.0)
                    + x1_ref[...].astype(jnp.float32))


def _bn_affine(stats, gamma, beta, count):
    mean = jnp.sum(stats[:, 0, :], axis=0) / count
    ex2 = jnp.sum(stats[:, 1, :], axis=0) / count
    var = jnp.maximum(ex2 - mean * mean, 0.0)
    scale = gamma * lax.rsqrt(var + _EPS)
    shift = beta - mean * scale
    C = gamma.shape[0]
    return (scale.reshape(1, C).astype(jnp.float32),
            shift.reshape(1, C).astype(jnp.float32))


def kernel(x, w1, b1, g1, be1, w2, b2, g2, be2):
    N, C, H, W = x.shape
    K = w1.shape[-1]
    pad = (K - 1) // 2
    Wq = ((W + 2 * pad + 7) // 8) * 8
    count = N * H * W
    B = 2 if N % 2 == 0 else 1
    nb = N // B

    x_nhwc = jnp.transpose(x, (0, 2, 3, 1)).astype(jnp.float32)
    w1f = jnp.transpose(w1, (2, 1, 3, 0)).reshape(K * C, K * C).astype(
        jnp.bfloat16)
    w2f = jnp.transpose(w2, (2, 1, 3, 0)).reshape(K * C, K * C).astype(
        jnp.bfloat16)

    cparams = pltpu.CompilerParams(
        dimension_semantics=("parallel",),
        vmem_limit_bytes=_VMEM_LIMIT)
    conv_flops = 2 * N * H * Wq * K * C * K * C

    y1, st1 = pl.pallas_call(
        functools.partial(_conv1_kernel, B=B, K=K, H=H, W=W, Wq=Wq, C=C),
        out_shape=[jax.ShapeDtypeStruct((N, H, W, C), jnp.bfloat16),
                   jax.ShapeDtypeStruct((nb, 2, C), jnp.float32)],
        grid=(nb,),
        in_specs=[pl.BlockSpec((B, H, W, C), lambda n: (n, 0, 0, 0)),
                  pl.BlockSpec((K * C, K * C), lambda n: (0, 0))],
        out_specs=[pl.BlockSpec((B, H, W, C), lambda n: (n, 0, 0, 0)),
                   pl.BlockSpec((1, 2, C), lambda n: (n, 0, 0))],
        scratch_shapes=[pltpu.VMEM((B, H, W, C), jnp.bfloat16),
                        pltpu.VMEM((B, H, Wq, K * C), jnp.bfloat16),
                        pltpu.VMEM((B, H, Wq, K * C), jnp.float32)],
        compiler_params=cparams,
        cost_estimate=pl.CostEstimate(
            flops=conv_flops, transcendentals=0,
            bytes_accessed=N * H * W * C * 6),
    )(x_nhwc, w1f)

    s1, t1 = _bn_affine(st1, g1, be1, count)

    y2, x1, st2 = pl.pallas_call(
        functools.partial(_mid_kernel, B=B, K=K, H=H, W=W, Wq=Wq, C=C),
        out_shape=[jax.ShapeDtypeStruct((N, H, W, C), jnp.bfloat16),
                   jax.ShapeDtypeStruct((N, H, W, C), jnp.bfloat16),
                   jax.ShapeDtypeStruct((nb, 2, C), jnp.float32)],
        grid=(nb,),
        in_specs=[pl.BlockSpec((B, H, W, C), lambda n: (n, 0, 0, 0)),
                  pl.BlockSpec((B, H, W, C), lambda n: (n, 0, 0, 0)),
                  pl.BlockSpec((K * C, K * C), lambda n: (0, 0)),
                  pl.BlockSpec((1, C), lambda n: (0, 0)),
                  pl.BlockSpec((1, C), lambda n: (0, 0))],
        out_specs=[pl.BlockSpec((B, H, W, C), lambda n: (n, 0, 0, 0)),
                   pl.BlockSpec((B, H, W, C), lambda n: (n, 0, 0, 0)),
                   pl.BlockSpec((1, 2, C), lambda n: (n, 0, 0))],
        scratch_shapes=[pltpu.VMEM((B, H, Wq, K * C), jnp.bfloat16),
                        pltpu.VMEM((B, H, Wq, K * C), jnp.float32)],
        compiler_params=cparams,
        cost_estimate=pl.CostEstimate(
            flops=conv_flops, transcendentals=0,
            bytes_accessed=N * H * W * C * 10),
    )(y1, x_nhwc, w2f, s1, t1)

    s2, t2 = _bn_affine(st2, g2, be2, count)

    out = pl.pallas_call(
        _final_kernel,
        out_shape=jax.ShapeDtypeStruct((N, H, W, C), jnp.float32),
        grid=(nb,),
        in_specs=[pl.BlockSpec((B, H, W, C), lambda n: (n, 0, 0, 0)),
                  pl.BlockSpec((B, H, W, C), lambda n: (n, 0, 0, 0)),
                  pl.BlockSpec((1, C), lambda n: (0, 0)),
                  pl.BlockSpec((1, C), lambda n: (0, 0))],
        out_specs=pl.BlockSpec((B, H, W, C), lambda n: (n, 0, 0, 0)),
        compiler_params=cparams,
        cost_estimate=pl.CostEstimate(
            flops=3 * N * H * W * C, transcendentals=0,
            bytes_accessed=2 * N * H * W * C * 4),
    )(y2, x1, s2, t2)

    return jnp.transpose(out, (0, 3, 1, 2))


# speedup vs baseline: 1.0226x; 1.0226x over previous
import functools

import jax
import jax.numpy as jnp
from jax import lax
from jax.experimental import pallas as pl
from jax.experimental.pallas import tpu as pltpu

_EPS = 1e-5
_VMEM_LIMIT = 56 * 1024 * 1024


def _build_taps(src, col_ref, b, K, H, W, Wq, C, cast):
    pad = (K - 1) // 2
    col_ref[b, :, 0:pad, :] = jnp.zeros((H, pad, K * C), jnp.bfloat16)
    col_ref[b, :, pad + W:Wq, :] = jnp.zeros((H, Wq - pad - W, K * C),
                                             jnp.bfloat16)
    for di in range(K):
        r0 = max(0, pad - di)
        r1 = min(H, H + pad - di)
        if r0 > 0:
            col_ref[b, 0:r0, pad:pad + W, di * C:(di + 1) * C] = jnp.zeros(
                (r0, W, C), jnp.bfloat16)
        if r1 < H:
            col_ref[b, r1:H, pad:pad + W, di * C:(di + 1) * C] = jnp.zeros(
                (H - r1, W, C), jnp.bfloat16)
        patch = src[r0 + di - pad:r1 + di - pad]
        if cast:
            patch = patch.astype(jnp.bfloat16)
        col_ref[b, r0:r1, pad:pad + W, di * C:(di + 1) * C] = patch


def _conv_epilogue(col_ref, w_ref, p_ref, B, K, H, W, Wq, C):
    p_ref[...] = jnp.dot(
        col_ref[...].reshape(B * H * Wq, K * C), w_ref[...],
        preferred_element_type=jnp.float32).reshape(B, H, Wq, K * C)
    y = p_ref[:, :, 0:W, 0:C]
    for dj in range(1, K):
        y = y + p_ref[:, :, dj:dj + W, dj * C:(dj + 1) * C]
    return y


def _tile_stats(y, stat_ref, m_tile, C):
    acc = y.reshape(m_tile, C)
    stat_ref[0, 0:1, :] = jnp.sum(acc, axis=0, keepdims=True)
    stat_ref[0, 1:2, :] = jnp.sum(acc * acc, axis=0, keepdims=True)


def _conv1_kernel(x_ref, w_ref, y_ref, stat_ref, col_ref, p_ref,
                  *, B, K, H, W, Wq, C):
    for b in range(B):
        _build_taps(x_ref[b], col_ref, b, K, H, W, Wq, C, cast=True)
    y = _conv_epilogue(col_ref, w_ref, p_ref, B, K, H, W, Wq, C)
    y_ref[...] = y.astype(jnp.bfloat16)
    _tile_stats(y, stat_ref, B * H * W, C)


def _mid_kernel(y1_ref, x_ref, w_ref, s1_ref, t1_ref,
                y2_ref, x1_ref, stat_ref, col_ref, p_ref,
                *, B, K, H, W, Wq, C):
    x1 = (jnp.maximum(y1_ref[...].astype(jnp.float32) * s1_ref[...]
                      + t1_ref[...], 0.0) + x_ref[...])
    x1b = x1.astype(jnp.bfloat16)
    x1_ref[...] = x1b
    for b in range(B):
        _build_taps(x1b[b], col_ref, b, K, H, W, Wq, C, cast=False)
    y = _conv_epilogue(col_ref, w_ref, p_ref, B, K, H, W, Wq, C)
    y2_ref[...] = y.astype(jnp.bfloat16)
    _tile_stats(y, stat_ref, B * H * W, C)


def _final_kernel(y2_ref, x1_ref, s2_ref, t2_ref, out_ref):
    out_ref[...] = (jnp.maximum(y2_ref[...].astype(jnp.float32) * s2_ref[...]
                                + t2_ref[...], 0.0)
                    + x1_ref[...].astype(jnp.float32))


def _bn_affine(stats, gamma, beta, count):
    mean = jnp.sum(stats[:, 0, :], axis=0) / count
    ex2 = jnp.sum(stats[:, 1, :], axis=0) / count
    var = jnp.maximum(ex2 - mean * mean, 0.0)
    scale = gamma * lax.rsqrt(var + _EPS)
    shift = beta - mean * scale
    C = gamma.shape[0]
    return (scale.reshape(1, C).astype(jnp.float32),
            shift.reshape(1, C).astype(jnp.float32))


def kernel(x, w1, b1, g1, be1, w2, b2, g2, be2):
    N, C, H, W = x.shape
    K = w1.shape[-1]
    pad = (K - 1) // 2
    Wq = ((W + 2 * pad + 7) // 8) * 8
    count = N * H * W
    B = 2 if N % 2 == 0 else 1
    nb = N // B

    x_nhwc = jnp.transpose(x, (0, 2, 3, 1)).astype(jnp.float32)
    w1f = jnp.transpose(w1, (2, 1, 3, 0)).reshape(K * C, K * C).astype(
        jnp.bfloat16)
    w2f = jnp.transpose(w2, (2, 1, 3, 0)).reshape(K * C, K * C).astype(
        jnp.bfloat16)

    cparams = pltpu.CompilerParams(
        dimension_semantics=("parallel",),
        vmem_limit_bytes=_VMEM_LIMIT)
    conv_flops = 2 * N * H * Wq * K * C * K * C

    y1, st1 = pl.pallas_call(
        functools.partial(_conv1_kernel, B=B, K=K, H=H, W=W, Wq=Wq, C=C),
        out_shape=[jax.ShapeDtypeStruct((N, H, W, C), jnp.bfloat16),
                   jax.ShapeDtypeStruct((nb, 2, C), jnp.float32)],
        grid=(nb,),
        in_specs=[pl.BlockSpec((B, H, W, C), lambda n: (n, 0, 0, 0)),
                  pl.BlockSpec((K * C, K * C), lambda n: (0, 0))],
        out_specs=[pl.BlockSpec((B, H, W, C), lambda n: (n, 0, 0, 0)),
                   pl.BlockSpec((1, 2, C), lambda n: (n, 0, 0))],
        scratch_shapes=[pltpu.VMEM((B, H, Wq, K * C), jnp.bfloat16),
                        pltpu.VMEM((B, H, Wq, K * C), jnp.float32)],
        compiler_params=cparams,
        cost_estimate=pl.CostEstimate(
            flops=conv_flops, transcendentals=0,
            bytes_accessed=N * H * W * C * 6),
    )(x_nhwc, w1f)

    s1, t1 = _bn_affine(st1, g1, be1, count)

    y2, x1, st2 = pl.pallas_call(
        functools.partial(_mid_kernel, B=B, K=K, H=H, W=W, Wq=Wq, C=C),
        out_shape=[jax.ShapeDtypeStruct((N, H, W, C), jnp.bfloat16),
                   jax.ShapeDtypeStruct((N, H, W, C), jnp.bfloat16),
                   jax.ShapeDtypeStruct((nb, 2, C), jnp.float32)],
        grid=(nb,),
        in_specs=[pl.BlockSpec((B, H, W, C), lambda n: (n, 0, 0, 0)),
                  pl.BlockSpec((B, H, W, C), lambda n: (n, 0, 0, 0)),
                  pl.BlockSpec((K * C, K * C), lambda n: (0, 0)),
                  pl.BlockSpec((1, C), lambda n: (0, 0)),
                  pl.BlockSpec((1, C), lambda n: (0, 0))],
        out_specs=[pl.BlockSpec((B, H, W, C), lambda n: (n, 0, 0, 0)),
                   pl.BlockSpec((B, H, W, C), lambda n: (n, 0, 0, 0)),
                   pl.BlockSpec((1, 2, C), lambda n: (n, 0, 0))],
        scratch_shapes=[pltpu.VMEM((B, H, Wq, K * C), jnp.bfloat16),
                        pltpu.VMEM((B, H, Wq, K * C), jnp.float32)],
        compiler_params=cparams,
        cost_estimate=pl.CostEstimate(
            flops=conv_flops, transcendentals=0,
            bytes_accessed=N * H * W * C * 10),
    )(y1, x_nhwc, w2f, s1, t1)

    s2, t2 = _bn_affine(st2, g2, be2, count)

    out = pl.pallas_call(
        _final_kernel,
        out_shape=jax.ShapeDtypeStruct((N, H, W, C), jnp.float32),
        grid=(nb,),
        in_specs=[pl.BlockSpec((B, H, W, C), lambda n: (n, 0, 0, 0)),
                  pl.BlockSpec((B, H, W, C), lambda n: (n, 0, 0, 0)),
                  pl.BlockSpec((1, C), lambda n: (0, 0)),
                  pl.BlockSpec((1, C), lambda n: (0, 0))],
        out_specs=pl.BlockSpec((B, H, W, C), lambda n: (n, 0, 0, 0)),
        compiler_params=cparams,
        cost_estimate=pl.CostEstimate(
            flops=3 * N * H * W * C, transcendentals=0,
            bytes_accessed=2 * N * H * W * C * 4),
    )(y2, x1, s2, t2)

    return jnp.transpose(out, (0, 3, 1, 2))
